# baseline (speedup 1.0000x reference)
import functools

import jax
import jax.numpy as jnp
from jax import lax
from jax.experimental import pallas as pl
from jax.experimental.pallas import tpu as pltpu
from jax.experimental.pallas import tpu_sc as plsc

INIT_SIZE = 8192
EMB_DIM = 1024
ORIGIN_SHIFT = INIT_SIZE // 2 + 1

NUM_SC_CORES = 2
NUM_SUBCORES = 16

CHUNK = 16
DEPTH_IN = 5
NBUF = 7
CHUNKS_C0 = 17
CHUNKS_C1 = 15


def _sc_row_range_copy(embed, n_rows, start_row):
    emb_dim = embed.shape[1]
    assert NUM_SUBCORES * CHUNK * (CHUNKS_C0 + CHUNKS_C1) == n_rows
    max_chunks = max(CHUNKS_C0, CHUNKS_C1)

    mesh = plsc.VectorSubcoreMesh(core_axis_name="c", subcore_axis_name="s")

    @functools.partial(
        pl.kernel,
        mesh=mesh,
        out_type=jax.ShapeDtypeStruct((n_rows, emb_dim), embed.dtype),
        scratch_types=(
            [pltpu.VMEM((CHUNK, emb_dim), embed.dtype) for _ in range(NBUF)]
            + [pltpu.VMEM((CHUNK,), jnp.int32) for _ in range(max_chunks)]
            + [pltpu.SemaphoreType.DMA for _ in range(2 * NBUF)]
        ),
    )
    def body(embed_hbm, out_hbm, *scratch):
        bufs = scratch[:NBUF]
        idxs = scratch[NBUF:NBUF + max_chunks]
        sin = scratch[NBUF + max_chunks:NBUF + max_chunks + NBUF]
        sout = scratch[NBUF + max_chunks + NBUF:]
        cid = lax.axis_index("c")
        sid = lax.axis_index("s")
        iota16 = lax.iota(jnp.int32, 16)

        def pipeline(base, n_chunks):
            for i in range(n_chunks):
                for k in range(CHUNK // 16):
                    idxs[i][pl.ds(16 * k, 16)] = (
                        iota16 + (base + start_row + i * CHUNK + 16 * k))

            def in_copy(i):
                return pltpu.make_async_copy(
                    embed_hbm.at[idxs[i]], bufs[i % NBUF], sin[i % NBUF])

            def out_copy(i):
                return pltpu.make_async_copy(
                    bufs[i % NBUF],
                    out_hbm.at[pl.ds(base + i * CHUNK, CHUNK)],
                    sout[i % NBUF])

            outs = [None] * n_chunks
            for i in range(min(DEPTH_IN, n_chunks)):
                in_copy(i).start()
            for i in range(n_chunks):
                in_copy(i).wait()
                oc = out_copy(i)
                oc.start()
                outs[i] = oc
                j = i + DEPTH_IN
                if j < n_chunks:
                    k = j - NBUF
                    if k >= 0:
                        outs[k].wait()
                        outs[k] = None
                    in_copy(j).start()
            for oc in outs:
                if oc is not None:
                    oc.wait()

        rows_c0 = CHUNKS_C0 * CHUNK

        @pl.when(cid == 0)
        def _():
            pipeline(sid * rows_c0, CHUNKS_C0)

        @pl.when(cid == 1)
        def _():
            pipeline(NUM_SUBCORES * rows_c0 + sid * (CHUNKS_C1 * CHUNK),
                     CHUNKS_C1)

    return body(embed)


def kernel(input, embed):
    bsz, seq_len = input.shape
    n_rows = 2 * seq_len
    start_row = ORIGIN_SHIFT - seq_len
    return _sc_row_range_copy(embed, n_rows, start_row)


# speedup vs baseline: 1.0575x; 1.0575x over previous
import functools

import jax
import jax.numpy as jnp
from jax import lax
from jax.experimental import pallas as pl
from jax.experimental.pallas import tpu as pltpu
from jax.experimental.pallas import tpu_sc as plsc

INIT_SIZE = 8192
EMB_DIM = 1024
ORIGIN_SHIFT = INIT_SIZE // 2 + 1

NUM_SC_CORES = 2
NUM_SUBCORES = 16
NUM_WORKERS = NUM_SC_CORES * NUM_SUBCORES

CHUNK = 16
DEPTH_IN = 5
NBUF = 7


def _sc_row_range_copy(embed, n_rows, start_row):
    emb_dim = embed.shape[1]
    rows_per_w = n_rows // NUM_WORKERS
    n_chunks = rows_per_w // CHUNK
    assert rows_per_w * NUM_WORKERS == n_rows
    assert n_chunks * CHUNK == rows_per_w
    assert CHUNK % 16 == 0

    mesh = plsc.VectorSubcoreMesh(core_axis_name="c", subcore_axis_name="s")

    @functools.partial(
        pl.kernel,
        mesh=mesh,
        out_type=jax.ShapeDtypeStruct((n_rows, emb_dim), embed.dtype),
        scratch_types=(
            [pltpu.VMEM((CHUNK, emb_dim), embed.dtype) for _ in range(NBUF)]
            + [pltpu.VMEM((CHUNK,), jnp.int32) for _ in range(n_chunks)]
            + [pltpu.SemaphoreType.DMA for _ in range(2 * NBUF)]
        ),
    )
    def body(embed_hbm, out_hbm, *scratch):
        bufs = scratch[:NBUF]
        idxs = scratch[NBUF:NBUF + n_chunks]
        sin = scratch[NBUF + n_chunks:NBUF + n_chunks + NBUF]
        sout = scratch[NBUF + n_chunks + NBUF:]
        wid = lax.axis_index("s") * NUM_SC_CORES + lax.axis_index("c")
        base = wid * rows_per_w

        iota16 = lax.iota(jnp.int32, 16)
        for i in range(n_chunks):
            for k in range(CHUNK // 16):
                idxs[i][pl.ds(16 * k, 16)] = (
                    iota16 + (base + start_row + i * CHUNK + 16 * k))

        def in_copy(i):
            return pltpu.make_async_copy(
                embed_hbm.at[idxs[i]], bufs[i % NBUF], sin[i % NBUF])

        def out_copy(i):
            return pltpu.make_async_copy(
                bufs[i % NBUF],
                out_hbm.at[pl.ds(base + i * CHUNK, CHUNK)],
                sout[i % NBUF])

        outs = [None] * n_chunks
        for i in range(min(DEPTH_IN, n_chunks)):
            in_copy(i).start()
        for i in range(n_chunks):
            in_copy(i).wait()
            oc = out_copy(i)
            oc.start()
            outs[i] = oc
            j = i + DEPTH_IN
            if j < n_chunks:
                k = j - NBUF
                if k >= 0:
                    outs[k].wait()
                    outs[k] = None
                in_copy(j).start()
        for oc in outs:
            if oc is not None:
                oc.wait()

    return body(embed)


def kernel(input, embed):
    bsz, seq_len = input.shape
    n_rows = 2 * seq_len
    start_row = ORIGIN_SHIFT - seq_len
    return _sc_row_range_copy(embed, n_rows, start_row)
